# Initial kernel scaffold; baseline (speedup 1.0000x reference)
import jax
import jax.numpy as jnp
from jax.experimental import pallas as pl


def kernel(x, pe_table):
    raise NotImplementedError("write your pallas kernel here")


# speedup vs baseline: 1.4581x; 1.4581x over previous
import jax
import jax.numpy as jnp
from jax.experimental import pallas as pl


def _body(x_ref, pe_ref, o_ref):
    o_ref[...] = x_ref[...] + pe_ref[...]


def kernel(x, pe_table):
    B, L, D = x.shape
    blk = 512
    return pl.pallas_call(
        _body,
        grid=(B, L // blk),
        in_specs=[
            pl.BlockSpec((1, blk, D), lambda b, i: (b, i, 0)),
            pl.BlockSpec((blk, D), lambda b, i: (i, 0)),
        ],
        out_specs=pl.BlockSpec((1, blk, D), lambda b, i: (b, i, 0)),
        out_shape=jax.ShapeDtypeStruct(x.shape, x.dtype),
    )(x, pe_table)


# speedup vs baseline: 1.6859x; 1.1563x over previous
import jax
import jax.numpy as jnp
from jax.experimental import pallas as pl


def _body(x_ref, pe_ref, o_ref):
    o_ref[...] = x_ref[...] + pe_ref[...]


def kernel(x, pe_table):
    B, L, D = x.shape
    blk = 512
    return pl.pallas_call(
        _body,
        grid=(L // blk, B),
        in_specs=[
            pl.BlockSpec((1, blk, D), lambda i, b: (b, i, 0)),
            pl.BlockSpec((blk, D), lambda i, b: (i, 0)),
        ],
        out_specs=pl.BlockSpec((1, blk, D), lambda i, b: (b, i, 0)),
        out_shape=jax.ShapeDtypeStruct(x.shape, x.dtype),
    )(x, pe_table)
